# Initial kernel scaffold; baseline (speedup 1.0000x reference)
import jax
import jax.numpy as jnp
from jax.experimental import pallas as pl


def kernel(indices, table):
    raise NotImplementedError("write your pallas kernel here")


# speedup vs baseline: 4.4660x; 4.4660x over previous
import functools

import jax
import jax.numpy as jnp
from jax import lax
from jax.experimental import pallas as pl
from jax.experimental.pallas import tpu as pltpu
from jax.experimental.pallas import tpu_sc as plsc

_V = 26
_L = 16
_NW = 32
_C = 2048


def _onehot_body(idx_hbm, out_hbm, idx_v, rows_v, chunks):
    wid = lax.axis_index("s") * 2 + lax.axis_index("c")
    base = wid * (chunks * _C)

    zeros = jnp.zeros((_L,), jnp.float32)
    ones = jnp.ones((_L,), jnp.float32)
    lane26 = lax.iota(jnp.int32, _L) * _V

    def zero_body(i, _):
        rows_v[pl.ds(i * _L, _L)] = zeros
        return 0

    lax.fori_loop(0, _C * _V // _L, zero_body, 0)

    U = 4

    def chunk_body(c, _):
        cb = base + c * _C
        pltpu.sync_copy(idx_hbm.at[pl.ds(cb, _C)], idx_v)

        def set_body(j, _):
            for u in range(U):
                jj = j * U + u
                v = idx_v[pl.ds(jj * _L, _L)]
                pos = jj * (_L * _V) + lane26 + v
                plsc.store_scatter(rows_v, [pos], ones)
            return 0

        lax.fori_loop(0, _C // (_L * U), set_body, 0)
        pltpu.sync_copy(rows_v, out_hbm.at[pl.ds(cb * _V, _C * _V)])

        def clr_body(j, _):
            for u in range(U):
                jj = j * U + u
                v = idx_v[pl.ds(jj * _L, _L)]
                pos = jj * (_L * _V) + lane26 + v
                plsc.store_scatter(rows_v, [pos], zeros)
            return 0

        lax.fori_loop(0, _C // (_L * U), clr_body, 0)
        return 0

    lax.fori_loop(0, chunks, chunk_body, 0)


def kernel(indices, table):
    B0, Lseq = indices.shape
    B = B0 * Lseq
    assert B % (_NW * _C) == 0
    chunks = B // (_NW * _C)

    idx = indices.reshape(B).astype(jnp.int32)
    mesh = plsc.VectorSubcoreMesh(core_axis_name="c", subcore_axis_name="s")

    k = functools.partial(
        pl.kernel,
        out_type=jax.ShapeDtypeStruct((B * _V,), jnp.float32),
        mesh=mesh,
        compiler_params=pltpu.CompilerParams(needs_layout_passes=False),
        scratch_types=[
            pltpu.VMEM((_C,), jnp.int32),
            pltpu.VMEM((_C * _V,), jnp.float32),
        ],
    )(functools.partial(_onehot_body, chunks=chunks))

    out = k(idx)
    return out.reshape(B0, Lseq, _V)


# speedup vs baseline: 47.6324x; 10.6655x over previous
import functools

import jax
import jax.numpy as jnp
from jax import lax
from jax.experimental import pallas as pl
from jax.experimental.pallas import tpu as pltpu
from jax.experimental.pallas import tpu_sc as plsc

_V = 26
_L = 16
_NW = 32
_K = 4
_TW = 1024


def _onehot_body(idx_hbm, out_hbm, idx_v, buf, tiles_per_w):
    wid = lax.axis_index("s") * 2 + lax.axis_index("c")
    t_base = wid * tiles_per_w

    zeros = jnp.zeros((_L,), jnp.float32)
    ones = jnp.ones((_L,), jnp.float32)
    lane = lax.iota(jnp.int32, _L)

    def zero_body(i, _):
        buf[i >> 8, (i >> 3) & 31, pl.ds((i & 7) * _L, _L)] = zeros
        return 0

    lax.fori_loop(0, _V * _K * _TW // _L, zero_body, 0)

    U = 4

    def chunk_body(c, _):
        t0 = t_base + c * _K
        pltpu.sync_copy(idx_hbm.at[pl.ds(t0 * _TW, _K * _TW)], idx_v)

        for k in range(_K):

            def set_body(j, _, k=k):
                for u in range(U):
                    jj = j * U + u
                    v = idx_v[pl.ds(k * _TW + jj * _L, _L)]
                    within = jj * _L + lane
                    row = (within >> 7) + (k * 8)
                    col = within & 127
                    plsc.store_scatter(buf, [v, row, col], ones)
                return 0

            lax.fori_loop(0, _TW // (_L * U), set_body, 0)

        pltpu.sync_copy(buf, out_hbm.at[:, pl.ds(t0 * 8, _K * 8), :])

        for k in range(_K):

            def clr_body(j, _, k=k):
                for u in range(U):
                    jj = j * U + u
                    v = idx_v[pl.ds(k * _TW + jj * _L, _L)]
                    within = jj * _L + lane
                    row = (within >> 7) + (k * 8)
                    col = within & 127
                    plsc.store_scatter(buf, [v, row, col], zeros)
                return 0

            lax.fori_loop(0, _TW // (_L * U), clr_body, 0)
        return 0

    lax.fori_loop(0, tiles_per_w // _K, chunk_body, 0)


def kernel(indices, table):
    B0, Lseq = indices.shape
    B = B0 * Lseq
    n_tiles = B // _TW
    assert Lseq % 8 == 0 and B0 % 128 == 0 and n_tiles % (_NW * _K) == 0
    tiles_per_w = n_tiles // _NW

    idx = (
        indices.astype(jnp.int32)
        .transpose(1, 0)
        .reshape(Lseq // 8, 8, B0 // 128, 128)
        .transpose(0, 2, 1, 3)
        .reshape(B)
    )

    mesh = plsc.VectorSubcoreMesh(core_axis_name="c", subcore_axis_name="s")
    k = functools.partial(
        pl.kernel,
        out_type=jax.ShapeDtypeStruct((_V, n_tiles * 8, 128), jnp.float32),
        mesh=mesh,
        compiler_params=pltpu.CompilerParams(needs_layout_passes=False),
        scratch_types=[
            pltpu.VMEM((_K * _TW,), jnp.int32),
            pltpu.VMEM((_V, _K * 8, 128), jnp.float32),
        ],
    )(functools.partial(_onehot_body, tiles_per_w=tiles_per_w))

    out = k(idx)
    return (
        out.reshape(_V, Lseq // 8, B0 // 128, 8, 128)
        .transpose(2, 4, 1, 3, 0)
        .reshape(B0, Lseq, _V)
    )


# speedup vs baseline: 65.2961x; 1.3708x over previous
import functools

import jax
import jax.numpy as jnp
from jax import lax
from jax.experimental import pallas as pl
from jax.experimental.pallas import tpu as pltpu
from jax.experimental.pallas import tpu_sc as plsc

_V = 26
_L = 16
_NW = 32
_K = 2
_TW = 1024
_U = 4


def _scatter_chunk(idx_v, buf, val):
    lane = lax.iota(jnp.int32, _L)
    for k in range(_K):

        def body(j, _, k=k):
            for u in range(_U):
                jj = j * _U + u
                v = idx_v[pl.ds(k * _TW + jj * _L, _L)]
                within = jj * _L + lane
                row = (within >> 7) + (k * 8)
                col = within & 127
                plsc.store_scatter(buf, [v, row, col], val)
            return 0

        lax.fori_loop(0, _TW // (_L * _U), body, 0)


def _onehot_body(idx_hbm, out_hbm, idx0, idx1, buf0, buf1, sem0, sem1,
                 tiles_per_w):
    wid = lax.axis_index("s") * 2 + lax.axis_index("c")
    t_base = wid * tiles_per_w

    zeros = jnp.zeros((_L,), jnp.float32)
    ones = jnp.ones((_L,), jnp.float32)
    idx_b = (idx0, idx1)
    buf_b = (buf0, buf1)
    sem_b = (sem0, sem1)

    def zero_body(i, _):
        for buf in buf_b:
            buf[i >> 7, (i >> 3) & 15, pl.ds((i & 7) * _L, _L)] = zeros
        return 0

    lax.fori_loop(0, _V * _K * _TW // _L, zero_body, 0)

    def run_chunk(c, b):
        t0 = t_base + c * _K
        pltpu.sync_copy(idx_hbm.at[pl.ds(t0 * _TW, _K * _TW)], idx_b[b])
        _scatter_chunk(idx_b[b], buf_b[b], ones)
        pltpu.async_copy(buf_b[b], out_hbm.at[:, pl.ds(t0 * 8, _K * 8), :],
                         sem_b[b])

    for b in range(2):
        run_chunk(b, b)

    def loop_body(c2, _):
        for b in range(2):
            c = c2 * 2 + b
            pltpu.make_async_copy(
                buf_b[b], out_hbm.at[:, pl.ds(0, _K * 8), :], sem_b[b]
            ).wait()
            _scatter_chunk(idx_b[b], buf_b[b], zeros)
            run_chunk(c, b)
        return 0

    lax.fori_loop(1, tiles_per_w // _K // 2, loop_body, 0)

    for b in range(2):
        pltpu.make_async_copy(
            buf_b[b], out_hbm.at[:, pl.ds(0, _K * 8), :], sem_b[b]
        ).wait()


def kernel(indices, table):
    B0, Lseq = indices.shape
    B = B0 * Lseq
    n_tiles = B // _TW
    assert Lseq % 8 == 0 and B0 % 128 == 0 and n_tiles % (_NW * _K * 2) == 0
    tiles_per_w = n_tiles // _NW

    idx = (
        indices.astype(jnp.int32)
        .transpose(1, 0)
        .reshape(Lseq // 8, 8, B0 // 128, 128)
        .transpose(0, 2, 1, 3)
        .reshape(B)
    )

    mesh = plsc.VectorSubcoreMesh(core_axis_name="c", subcore_axis_name="s")
    k = functools.partial(
        pl.kernel,
        out_type=jax.ShapeDtypeStruct((_V, n_tiles * 8, 128), jnp.float32),
        mesh=mesh,
        compiler_params=pltpu.CompilerParams(needs_layout_passes=False),
        scratch_types=[
            pltpu.VMEM((_K * _TW,), jnp.int32),
            pltpu.VMEM((_K * _TW,), jnp.int32),
            pltpu.VMEM((_V, _K * 8, 128), jnp.float32),
            pltpu.VMEM((_V, _K * 8, 128), jnp.float32),
            pltpu.SemaphoreType.DMA,
            pltpu.SemaphoreType.DMA,
        ],
    )(functools.partial(_onehot_body, tiles_per_w=tiles_per_w))

    out = k(idx)
    return (
        out.reshape(_V, Lseq // 8, B0 // 128, 8, 128)
        .transpose(2, 4, 1, 3, 0)
        .reshape(B0, Lseq, _V)
    )


# speedup vs baseline: 66.4041x; 1.0170x over previous
import functools

import jax
import jax.numpy as jnp
from jax import lax
from jax.experimental import pallas as pl
from jax.experimental.pallas import tpu as pltpu
from jax.experimental.pallas import tpu_sc as plsc

_V = 26
_L = 16
_NW = 32
_K = 2
_TW = 1024
_U = 4


def _scatter_chunk(idx_v, buf, val):
    lane = lax.iota(jnp.int32, _L)
    for k in range(_K):

        def body(j, _, k=k):
            for u in range(_U):
                jj = j * _U + u
                v = idx_v[pl.ds(k * _TW + jj * _L, _L)]
                within = jj * _L + lane
                row = (within >> 7) + (k * 8)
                col = within & 127
                plsc.store_scatter(buf, [v, row, col], val)
            return 0

        lax.fori_loop(0, _TW // (_L * _U), body, 0)


def _onehot_body(idx_hbm, zeros_hbm, out_hbm, idx0, idx1, buf0, buf1,
                 sem0, sem1, tiles_per_w):
    wid = lax.axis_index("s") * 2 + lax.axis_index("c")
    t_base = wid * tiles_per_w

    zeros = jnp.zeros((_L,), jnp.float32)
    ones = jnp.ones((_L,), jnp.float32)
    idx_b = (idx0, idx1)
    buf_b = (buf0, buf1)
    sem_b = (sem0, sem1)

    for b in range(2):
        pltpu.async_copy(zeros_hbm, buf_b[b], sem_b[b])

    def run_chunk(c, b):
        t0 = t_base + c * _K
        pltpu.sync_copy(idx_hbm.at[pl.ds(t0 * _TW, _K * _TW)], idx_b[b])
        _scatter_chunk(idx_b[b], buf_b[b], ones)
        pltpu.async_copy(buf_b[b], out_hbm.at[:, pl.ds(t0 * 8, _K * 8), :],
                         sem_b[b])

    for b in range(2):
        pltpu.make_async_copy(zeros_hbm, buf_b[b], sem_b[b]).wait()
        run_chunk(b, b)

    def loop_body(c2, _):
        for b in range(2):
            c = c2 * 2 + b
            pltpu.make_async_copy(
                buf_b[b], out_hbm.at[:, pl.ds(0, _K * 8), :], sem_b[b]
            ).wait()
            _scatter_chunk(idx_b[b], buf_b[b], zeros)
            run_chunk(c, b)
        return 0

    lax.fori_loop(1, tiles_per_w // _K // 2, loop_body, 0)

    for b in range(2):
        pltpu.make_async_copy(
            buf_b[b], out_hbm.at[:, pl.ds(0, _K * 8), :], sem_b[b]
        ).wait()


def kernel(indices, table):
    B0, Lseq = indices.shape
    B = B0 * Lseq
    n_tiles = B // _TW
    assert Lseq % 8 == 0 and B0 % 128 == 0 and n_tiles % (_NW * _K * 2) == 0
    tiles_per_w = n_tiles // _NW

    idx = (
        indices.astype(jnp.int32)
        .transpose(1, 0)
        .reshape(Lseq // 8, 8, B0 // 128, 128)
        .transpose(0, 2, 1, 3)
        .reshape(B)
    )

    mesh = plsc.VectorSubcoreMesh(core_axis_name="c", subcore_axis_name="s")
    k = functools.partial(
        pl.kernel,
        out_type=jax.ShapeDtypeStruct((_V, n_tiles * 8, 128), jnp.float32),
        mesh=mesh,
        compiler_params=pltpu.CompilerParams(needs_layout_passes=False),
        scratch_types=[
            pltpu.VMEM((_K * _TW,), jnp.int32),
            pltpu.VMEM((_K * _TW,), jnp.int32),
            pltpu.VMEM((_V, _K * 8, 128), jnp.float32),
            pltpu.VMEM((_V, _K * 8, 128), jnp.float32),
            pltpu.SemaphoreType.DMA,
            pltpu.SemaphoreType.DMA,
        ],
    )(functools.partial(_onehot_body, tiles_per_w=tiles_per_w))

    out = k(idx, jnp.zeros((_V, _K * 8, 128), jnp.float32))
    return (
        out.reshape(_V, Lseq // 8, B0 // 128, 8, 128)
        .transpose(2, 4, 1, 3, 0)
        .reshape(B0, Lseq, _V)
    )


# speedup vs baseline: 68.7099x; 1.0347x over previous
import functools

import jax
import jax.numpy as jnp
from jax import lax
from jax.experimental import pallas as pl
from jax.experimental.pallas import tpu as pltpu
from jax.experimental.pallas import tpu_sc as plsc

_V = 26
_L = 16
_NW = 32
_K = 2
_TW = 1024
_U = 8


def _scatter_chunk(idx_v, buf, val):
    lane = lax.iota(jnp.int32, _L)
    for k in range(_K):

        def body(j, _, k=k):
            row = jnp.full((_L,), k * 8, jnp.int32) + j
            for u in range(_U):
                jj = j * _U + u
                v = idx_v[pl.ds(k * _TW + jj * _L, _L)]
                col = u * _L + lane
                plsc.store_scatter(buf, [v, row, col], val)
            return 0

        lax.fori_loop(0, _TW // (_L * _U), body, 0)


def _onehot_body(idx_hbm, zeros_hbm, out_hbm, idx0, idx1, buf0, buf1,
                 sem0, sem1, tiles_per_w):
    wid = lax.axis_index("s") * 2 + lax.axis_index("c")
    t_base = wid * tiles_per_w

    zeros = jnp.zeros((_L,), jnp.float32)
    ones = jnp.ones((_L,), jnp.float32)
    idx_b = (idx0, idx1)
    buf_b = (buf0, buf1)
    sem_b = (sem0, sem1)

    for b in range(2):
        pltpu.async_copy(zeros_hbm, buf_b[b], sem_b[b])

    def run_chunk(c, b):
        t0 = t_base + c * _K
        pltpu.sync_copy(idx_hbm.at[pl.ds(t0 * _TW, _K * _TW)], idx_b[b])
        _scatter_chunk(idx_b[b], buf_b[b], ones)
        pltpu.async_copy(buf_b[b], out_hbm.at[:, pl.ds(t0 * 8, _K * 8), :],
                         sem_b[b])

    for b in range(2):
        pltpu.make_async_copy(zeros_hbm, buf_b[b], sem_b[b]).wait()
        run_chunk(b, b)

    def loop_body(c2, _):
        for b in range(2):
            c = c2 * 2 + b
            pltpu.make_async_copy(
                buf_b[b], out_hbm.at[:, pl.ds(0, _K * 8), :], sem_b[b]
            ).wait()
            _scatter_chunk(idx_b[b], buf_b[b], zeros)
            run_chunk(c, b)
        return 0

    lax.fori_loop(1, tiles_per_w // _K // 2, loop_body, 0)

    for b in range(2):
        pltpu.make_async_copy(
            buf_b[b], out_hbm.at[:, pl.ds(0, _K * 8), :], sem_b[b]
        ).wait()


def kernel(indices, table):
    B0, Lseq = indices.shape
    B = B0 * Lseq
    n_tiles = B // _TW
    assert Lseq % 8 == 0 and B0 % 128 == 0 and n_tiles % (_NW * _K * 2) == 0
    tiles_per_w = n_tiles // _NW

    idx = (
        indices.astype(jnp.int32)
        .transpose(1, 0)
        .reshape(Lseq // 8, 8, B0 // 128, 128)
        .transpose(0, 2, 1, 3)
        .reshape(B)
    )

    mesh = plsc.VectorSubcoreMesh(core_axis_name="c", subcore_axis_name="s")
    k = functools.partial(
        pl.kernel,
        out_type=jax.ShapeDtypeStruct((_V, n_tiles * 8, 128), jnp.float32),
        mesh=mesh,
        compiler_params=pltpu.CompilerParams(needs_layout_passes=False),
        scratch_types=[
            pltpu.VMEM((_K * _TW,), jnp.int32),
            pltpu.VMEM((_K * _TW,), jnp.int32),
            pltpu.VMEM((_V, _K * 8, 128), jnp.float32),
            pltpu.VMEM((_V, _K * 8, 128), jnp.float32),
            pltpu.SemaphoreType.DMA,
            pltpu.SemaphoreType.DMA,
        ],
    )(functools.partial(_onehot_body, tiles_per_w=tiles_per_w))

    out = k(idx, jnp.zeros((_V, _K * 8, 128), jnp.float32))
    return (
        out.reshape(_V, Lseq // 8, B0 // 128, 8, 128)
        .transpose(2, 4, 1, 3, 0)
        .reshape(B0, Lseq, _V)
    )
